# Initial kernel scaffold; baseline (speedup 1.0000x reference)
import jax
import jax.numpy as jnp
from jax.experimental import pallas as pl


def kernel(node_feat, edge_feat, edge_index, mlp_node, mlp_edge, gnn1_mlp, gnn2_mlp, gnn2_mlp_edge, graph_W, graph_b):
    raise NotImplementedError("write your pallas kernel here")


# speedup vs baseline: 2.2006x; 2.2006x over previous
import functools

import jax
import jax.numpy as jnp
from jax import lax
from jax.experimental import pallas as pl
from jax.experimental.pallas import tpu as pltpu
from jax.experimental.pallas import tpu_sc as plsc

N = 10000
E = 320000
H = 128

_NC = 2
_NS = 16
_NW = _NC * _NS
_EW = E // _NW

_GC = 1000
_SCC = 200
_RPT = 624
_RTAIL = N - _NS * _RPT


def _leaky(x):
    return jnp.where(x >= 0, x, 0.1 * x)


def _dot(a, b):
    return jnp.dot(a, b, preferred_element_type=jnp.float32)


def _node_pre_body(x, w1, b1, w2, b2, wa, o_ref):
    h = _leaky(_dot(x[...], w1[...]) + b1[...])
    nf = _leaky(_dot(h, w2[...]) + b2[...])
    o_ref[...] = _dot(nf, wa[...])


def _node_pre(node_feat, w1, b1, w2, b2, wa):
    bn = 1000
    return pl.pallas_call(
        _node_pre_body,
        grid=(N // bn,),
        in_specs=[
            pl.BlockSpec((bn, 128), lambda i: (i, 0)),
            pl.BlockSpec((128, 64), lambda i: (0, 0)),
            pl.BlockSpec((1, 64), lambda i: (0, 0)),
            pl.BlockSpec((64, 128), lambda i: (0, 0)),
            pl.BlockSpec((1, 128), lambda i: (0, 0)),
            pl.BlockSpec((128, 64), lambda i: (0, 0)),
        ],
        out_specs=pl.BlockSpec((bn, 64), lambda i: (i, 0)),
        out_shape=jax.ShapeDtypeStruct((N, 64), jnp.float32),
    )(node_feat, w1, b1, w2, b2, wa)


def _edge_fused_body(x, g1, w1e, b1e, w2e, b2e, we1, be1, we2, be2,
                     wb1, bb1, w21, b21, wb2, bb2,
                     ef_ref, e2_ref, m1_ref):
    h = _leaky(_dot(x[...], w1e[...]) + b1e[...])
    ef = _leaky(_dot(h, w2e[...]) + b2e[...])
    ef_ref[...] = ef
    h2 = _leaky(_dot(ef, we1[...]) + be1[...])
    ef2 = _leaky(_dot(h2, we2[...]) + be2[...])
    e2_ref[...] = _dot(ef2, wb2[...]) + bb2[...]
    hm = _leaky(g1[...] + _dot(ef, wb1[...]) + bb1[...])
    m1_ref[...] = _leaky(_dot(hm, w21[...]) + b21[...])


def _edge_fused(edge_feat, g1, w1e, b1e, w2e, b2e, we1, be1, we2, be2,
                wb1, bb1, w21, b21, wb2, bb2):
    be = 2000
    full = lambda r, c: pl.BlockSpec((r, c), lambda i: (0, 0))
    return pl.pallas_call(
        _edge_fused_body,
        grid=(E // be,),
        in_specs=[
            pl.BlockSpec((be, 16), lambda i: (i, 0)),
            pl.BlockSpec((be, 64), lambda i: (i, 0)),
            full(16, 64), full(1, 64), full(64, 128), full(1, 128),
            full(128, 64), full(1, 64), full(64, 128), full(1, 128),
            full(128, 64), full(1, 64), full(64, 128), full(1, 128),
            full(128, 64), full(1, 64),
        ],
        out_specs=[
            pl.BlockSpec((be, 128), lambda i: (i, 0)),
            pl.BlockSpec((be, 64), lambda i: (i, 0)),
            pl.BlockSpec((be, 128), lambda i: (i, 0)),
        ],
        out_shape=[
            jax.ShapeDtypeStruct((E, 128), jnp.float32),
            jax.ShapeDtypeStruct((E, 64), jnp.float32),
            jax.ShapeDtypeStruct((E, 128), jnp.float32),
        ],
    )(edge_feat, g1, w1e, b1e, w2e, b2e, we1, be1, we2, be2,
      wb1, bb1, w21, b21, wb2, bb2)


def _msg2_body(g2, e2, w22, b22, m_ref):
    hm = _leaky(g2[...] + e2[...])
    m_ref[...] = _leaky(_dot(hm, w22[...]) + b22[...])


def _msg2(g2, e2, w22, b22):
    be = 2000
    return pl.pallas_call(
        _msg2_body,
        grid=(E // be,),
        in_specs=[
            pl.BlockSpec((be, 64), lambda i: (i, 0)),
            pl.BlockSpec((be, 64), lambda i: (i, 0)),
            pl.BlockSpec((64, 128), lambda i: (0, 0)),
            pl.BlockSpec((1, 128), lambda i: (0, 0)),
        ],
        out_specs=pl.BlockSpec((be, 128), lambda i: (i, 0)),
        out_shape=jax.ShapeDtypeStruct((E, 128), jnp.float32),
    )(g2, e2, w22, b22)


def _node_pre2_body(p, wa, o_ref):
    x = p[0] + p[1]
    o_ref[...] = _dot(x, wa[...])


def _node_pre2(p, wa):
    bn = 1000
    return pl.pallas_call(
        _node_pre2_body,
        grid=(N // bn,),
        in_specs=[
            pl.BlockSpec((2, bn, 128), lambda i: (0, i, 0)),
            pl.BlockSpec((128, 64), lambda i: (0, 0)),
        ],
        out_specs=pl.BlockSpec((bn, 64), lambda i: (i, 0)),
        out_shape=jax.ShapeDtypeStruct((N, 64), jnp.float32),
    )(p, wa)


def _graph_body(xq, w, b, o_ref):
    k = pl.program_id(0)

    @pl.when(k == 0)
    def _():
        o_ref[...] = jnp.zeros_like(o_ref)

    x = xq[0:1, :] + xq[1:2, :]
    o_ref[...] += _dot(x, w[...])

    @pl.when(k == pl.num_programs(0) - 1)
    def _():
        o_ref[...] = _leaky(o_ref[...] + b[...])


def _graph_linear(q2, graph_W, graph_b):
    bk = 12800
    k_total = N * H
    return pl.pallas_call(
        _graph_body,
        grid=(k_total // bk,),
        in_specs=[
            pl.BlockSpec((2, bk), lambda k: (0, k)),
            pl.BlockSpec((bk, 128), lambda k: (k, 0)),
            pl.BlockSpec((1, 128), lambda k: (0, 0)),
        ],
        out_specs=pl.BlockSpec((1, 128), lambda k: (0, 0)),
        out_shape=jax.ShapeDtypeStruct((1, 128), jnp.float32),
    )(q2, graph_W, graph_b)


def _sc_gather(table, idx):
    mesh = plsc.VectorSubcoreMesh(core_axis_name="c", subcore_axis_name="s")

    @functools.partial(
        pl.kernel,
        out_type=jax.ShapeDtypeStruct((E, 64), jnp.float32),
        mesh=mesh,
        scratch_types=[
            pltpu.VMEM((_GC,), jnp.int32),
            pltpu.VMEM((_GC, 64), jnp.float32),
            pltpu.SemaphoreType.DMA,
        ],
        compiler_params=pltpu.CompilerParams(use_tc_tiling_on_sc=False),
    )
    def k(table_hbm, idx_hbm, out_hbm, idx_v, rows_v, sem):
        wid = lax.axis_index("s") * _NC + lax.axis_index("c")
        base = wid * _EW

        def body(i, carry):
            off = base + i * _GC
            pltpu.sync_copy(idx_hbm.at[pl.ds(off, _GC)], idx_v)
            pltpu.async_copy(table_hbm.at[idx_v], rows_v, sem).wait()
            pltpu.sync_copy(rows_v, out_hbm.at[pl.ds(off, _GC)])
            return carry

        lax.fori_loop(0, _EW // _GC, body, 0)

    return k(table, idx)


def _sc_scatter(m, idx, zrows):
    mesh = plsc.VectorSubcoreMesh(core_axis_name="c", subcore_axis_name="s")

    @functools.partial(
        pl.kernel,
        out_type=jax.ShapeDtypeStruct((_NC, N, H), jnp.float32),
        mesh=mesh,
        scratch_types=[
            pltpu.VMEM((_SCC,), jnp.int32),
            pltpu.VMEM((_SCC, H), jnp.float32),
            pltpu.VMEM_SHARED((N, H), jnp.float32),
        ],
    )
    def k(m_hbm, idx_hbm, z_hbm, out_hbm, idx_v, rows_v, acc_sh):
        c = lax.axis_index("c")
        s = lax.axis_index("s")
        wid = s * _NC + c
        def zbody(j, carry):
            pltpu.sync_copy(z_hbm, acc_sh.at[pl.ds(s * _RPT + j * 16, 16)])
            return carry

        lax.fori_loop(0, _RPT // 16, zbody, 0)

        @pl.when(s == _NS - 1)
        def _():
            pltpu.sync_copy(z_hbm, acc_sh.at[pl.ds(_NS * _RPT, _RTAIL)])

        plsc.subcore_barrier()

        base = wid * _EW

        def body(i, carry):
            off = base + i * _SCC
            pltpu.sync_copy(idx_hbm.at[pl.ds(off, _SCC)], idx_v)
            pltpu.sync_copy(m_hbm.at[pl.ds(off, _SCC)], rows_v)
            pltpu.sync_copy(rows_v, acc_sh.at[idx_v], add=True)
            return carry

        lax.fori_loop(0, _EW // _SCC, body, 0)
        plsc.subcore_barrier()
        pltpu.sync_copy(acc_sh.at[pl.ds(s * _RPT, _RPT)],
                        out_hbm.at[c, pl.ds(s * _RPT, _RPT)])

        @pl.when(s == _NS - 1)
        def _():
            pltpu.sync_copy(acc_sh.at[pl.ds(_NS * _RPT, _RTAIL)],
                            out_hbm.at[c, pl.ds(_NS * _RPT, _RTAIL)])

    return k(m, idx, zrows)


def kernel(node_feat, edge_feat, edge_index, mlp_node, mlp_edge,
           gnn1_mlp, gnn2_mlp, gnn2_mlp_edge, graph_W, graph_b):
    w1n, b1n, w2n, b2n = mlp_node
    w1e, b1e, w2e, b2e = mlp_edge
    w11, b11, w21, b21 = gnn1_mlp
    w12, b12, w22, b22 = gnn2_mlp
    we1, be1, we2, be2 = gnn2_mlp_edge

    r = lambda v: v.reshape(1, -1)
    wa1, wb1 = w11[:H], w11[H:]
    wa2, wb2 = w12[:H], w12[H:]

    dst = edge_index[1].astype(jnp.int32)
    zrows = jnp.zeros((16, H), jnp.float32)

    a1 = _node_pre(node_feat, w1n, r(b1n), w2n, r(b2n), wa1)
    g1 = _sc_gather(a1, dst)

    ef, e2, m1 = _edge_fused(edge_feat, g1, w1e, r(b1e), w2e, r(b2e),
                             we1, r(be1), we2, r(be2),
                             wb1, r(b11), w21, r(b21), wb2, r(b12))
    p = _sc_scatter(m1, dst, zrows)

    a2 = _node_pre2(p, wa2)
    g2 = _sc_gather(a2, dst)
    m2 = _msg2(g2, e2, w22, r(b22))
    q = _sc_scatter(m2, dst, zrows)

    g = _graph_linear(q.reshape(_NC, N * H), graph_W, r(graph_b))
    return (g.reshape(H), ef)
